# baseline (speedup 1.0000x reference)
import jax
import jax.numpy as jnp
import numpy as np
from jax import lax
from jax.experimental import pallas as pl
from jax.experimental.pallas import tpu as pltpu
from jax.experimental.pallas import tpu_sc as plsc

HIDDEN = 768
NUM_POS = 577
WIDTH = 24
N = 131072

NC = 2
NS = 16
NW = NC * NS
RPW = N // NW
C = 32
NCH = RPW // C

BIDX = 16384

_DENOM = np.float32(2.0 + 1e-6)


def _idx_body(pos_ref, idx_ref):
    p = pos_ref[...]
    f = (((p + 1.0) / _DENOM) * np.float32(WIDTH)).astype(jnp.int32)
    idx_ref[...] = f[:, 0] * WIDTH + f[:, 1]


def _sc_body(idx_hbm, table_hbm, out_hbm, table_sh, idx_v, rows0, rows1,
             g0, g1, s0, s1):
    sid = lax.axis_index("s")
    wid = sid * NC + lax.axis_index("c")
    base = wid * RPW
    rows = (rows0, rows1)
    gsem = (g0, g1)
    ssem = (s0, s1)

    @pl.when(sid == 0)
    def _stage_table():
        pltpu.sync_copy(table_hbm, table_sh)

    def _row(i):
        return table_sh.at[pl.ds(i * HIDDEN, HIDDEN)]

    pltpu.sync_copy(idx_hbm.at[pl.ds(base, RPW)], idx_v)

    plsc.subcore_barrier()

    def _fire_rows(b, ch):
        for s in range(C // 16):
            ivec = idx_v[pl.ds(ch * C + s * 16, 16)]
            for k in range(16):
                pltpu.async_copy(
                    _row(ivec[k]), rows[b].at[s * 16 + k], gsem[b]
                )

    def _drain_rows(b):
        pltpu.make_async_copy(
            out_hbm.at[pl.ds(base, C)], rows[b], gsem[b]
        ).wait()

    def _scatter(b, ch):
        pltpu.async_copy(
            rows[b], out_hbm.at[pl.ds(base + ch * C, C)], ssem[b]
        )

    def _wait_scatter(b, ch):
        pltpu.make_async_copy(
            rows[b], out_hbm.at[pl.ds(base + ch * C, C)], ssem[b]
        ).wait()

    _fire_rows(0, 0)

    @pl.loop(0, NCH, step=2)
    def _move(ch0):
        for b in range(2):
            ch = ch0 + b
            b1 = 1 - b
            nxt = ch + 1

            @pl.when(nxt < NCH)
            def _prefetch():
                @pl.when(nxt >= 2)
                def _reclaim():
                    _wait_scatter(b1, nxt - 2)

                _fire_rows(b1, nxt)

            _drain_rows(b)
            _scatter(b, ch)

    _wait_scatter((NCH - 2) % 2, NCH - 2)
    _wait_scatter((NCH - 1) % 2, NCH - 1)


@jax.jit
def _embed(pos, table_flat):
    idx = pl.pallas_call(
        _idx_body,
        grid=(N // BIDX,),
        in_specs=[pl.BlockSpec((BIDX, 2), lambda i: (i, 0))],
        out_specs=pl.BlockSpec((BIDX,), lambda i: (i,)),
        out_shape=jax.ShapeDtypeStruct((N,), jnp.int32),
    )(pos)

    mesh = plsc.VectorSubcoreMesh(
        core_axis_name="c", subcore_axis_name="s", num_cores=NC, num_subcores=NS
    )
    return pl.kernel(
        _sc_body,
        out_type=jax.ShapeDtypeStruct((N, HIDDEN), jnp.float32),
        mesh=mesh,
        scratch_types=[
            pltpu.VMEM_SHARED((NUM_POS * HIDDEN,), jnp.float32),
            pltpu.VMEM((RPW,), jnp.int32),
            pltpu.VMEM((C, HIDDEN), jnp.float32),
            pltpu.VMEM((C, HIDDEN), jnp.float32),
            pltpu.SemaphoreType.DMA,
            pltpu.SemaphoreType.DMA,
            pltpu.SemaphoreType.DMA,
            pltpu.SemaphoreType.DMA,
        ],
        compiler_params=pltpu.CompilerParams(needs_layout_passes=False),
    )(idx, table_flat)


def kernel(pos, positional_embeddings):
    table_flat = positional_embeddings.reshape(NUM_POS * HIDDEN)
    out = _embed(pos, table_flat)
    return out.reshape(1, N, HIDDEN)


# speedup vs baseline: 1.3843x; 1.3843x over previous
import jax
import jax.numpy as jnp
import numpy as np
from jax import lax
from jax.experimental import pallas as pl
from jax.experimental.pallas import tpu as pltpu
from jax.experimental.pallas import tpu_sc as plsc

HIDDEN = 768
NUM_POS = 577
WIDTH = 24
N = 131072

NC = 2
NS = 16
NW = NC * NS
RPW = N // NW
C = 32
NCH = RPW // C
NVEC = RPW // 16

_DENOM = np.float32(2.0 + 1e-6)


def _sc_body(pos_hbm, table_hbm, out_hbm, table_sh, pos_v, idx_v, rows0,
             rows1, g0, g1, s0, s1):
    sid = lax.axis_index("s")
    wid = sid * NC + lax.axis_index("c")
    base = wid * RPW
    rows = (rows0, rows1)
    gsem = (g0, g1)
    ssem = (s0, s1)

    @pl.when(sid == 0)
    def _stage_table():
        pltpu.sync_copy(table_hbm, table_sh)

    def _row(i):
        return table_sh.at[pl.ds(i * HIDDEN, HIDDEN)]

    pltpu.sync_copy(pos_hbm.at[pl.ds(base, RPW)], pos_v.at[pl.ds(0, RPW)])
    pltpu.sync_copy(
        pos_hbm.at[pl.ds(N + base, RPW)], pos_v.at[pl.ds(RPW, RPW)]
    )

    @pl.loop(0, NVEC)
    def _compute(v):
        xs = pos_v[pl.ds(v * 16, 16)]
        ys = pos_v[pl.ds(RPW + v * 16, 16)]
        fx = (((xs + 1.0) / _DENOM) * np.float32(WIDTH)).astype(jnp.int32)
        fy = (((ys + 1.0) / _DENOM) * np.float32(WIDTH)).astype(jnp.int32)
        idx_v[pl.ds(v * 16, 16)] = fx * WIDTH + fy

    plsc.subcore_barrier()

    def _fire_rows(b, ch):
        for s in range(C // 16):
            ivec = idx_v[pl.ds(ch * C + s * 16, 16)]
            for k in range(16):
                pltpu.async_copy(
                    _row(ivec[k]), rows[b].at[s * 16 + k], gsem[b]
                )

    def _drain_rows(b):
        pltpu.make_async_copy(
            out_hbm.at[pl.ds(base, C)], rows[b], gsem[b]
        ).wait()

    def _scatter(b, ch):
        pltpu.async_copy(
            rows[b], out_hbm.at[pl.ds(base + ch * C, C)], ssem[b]
        )

    def _wait_scatter(b, ch):
        pltpu.make_async_copy(
            rows[b], out_hbm.at[pl.ds(base + ch * C, C)], ssem[b]
        ).wait()

    _fire_rows(0, 0)

    @pl.loop(0, NCH, step=2)
    def _move(ch0):
        for b in range(2):
            ch = ch0 + b
            b1 = 1 - b
            nxt = ch + 1

            @pl.when(nxt < NCH)
            def _prefetch():
                @pl.when(nxt >= 2)
                def _reclaim():
                    _wait_scatter(b1, nxt - 2)

                _fire_rows(b1, nxt)

            _drain_rows(b)
            _scatter(b, ch)

    _wait_scatter((NCH - 2) % 2, NCH - 2)
    _wait_scatter((NCH - 1) % 2, NCH - 1)


@jax.jit
def _sc_embed(pos_tflat, table_flat):
    mesh = plsc.VectorSubcoreMesh(
        core_axis_name="c", subcore_axis_name="s", num_cores=NC, num_subcores=NS
    )
    return pl.kernel(
        _sc_body,
        out_type=jax.ShapeDtypeStruct((N, HIDDEN), jnp.float32),
        mesh=mesh,
        scratch_types=[
            pltpu.VMEM_SHARED((NUM_POS * HIDDEN,), jnp.float32),
            pltpu.VMEM((2 * RPW,), jnp.float32),
            pltpu.VMEM((RPW,), jnp.int32),
            pltpu.VMEM((C, HIDDEN), jnp.float32),
            pltpu.VMEM((C, HIDDEN), jnp.float32),
            pltpu.SemaphoreType.DMA,
            pltpu.SemaphoreType.DMA,
            pltpu.SemaphoreType.DMA,
            pltpu.SemaphoreType.DMA,
        ],
        compiler_params=pltpu.CompilerParams(needs_layout_passes=False),
    )(pos_tflat, table_flat)


def kernel(pos, positional_embeddings):
    pos_tflat = pos.T.reshape(2 * N)
    table_flat = positional_embeddings.reshape(NUM_POS * HIDDEN)
    out = _sc_embed(pos_tflat, table_flat)
    return out.reshape(1, N, HIDDEN)


# speedup vs baseline: 1.3954x; 1.0080x over previous
import jax
import jax.numpy as jnp
import numpy as np
from jax import lax
from jax.experimental import pallas as pl
from jax.experimental.pallas import tpu as pltpu
from jax.experimental.pallas import tpu_sc as plsc

HIDDEN = 768
NUM_POS = 577
WIDTH = 24
N = 131072

NC = 2
NS = 16
NW = NC * NS
RPW = N // NW
C = 32
NCH = RPW // C
NVEC = RPW // 16

_DENOM = np.float32(2.0 + 1e-6)


def _sc_body(pos_hbm, table_hbm, out_hbm, table_sh, pos_v, idx_v, rows0,
             rows1, g0, g1, s0, s1):
    sid = lax.axis_index("s")
    wid = sid * NC + lax.axis_index("c")
    base = wid * RPW
    rows = (rows0, rows1)
    gsem = (g0, g1)
    ssem = (s0, s1)

    @pl.when(sid == 0)
    def _stage_table():
        pltpu.sync_copy(table_hbm, table_sh)

    def _row(i):
        return table_sh.at[pl.ds(i * HIDDEN, HIDDEN)]

    pltpu.sync_copy(pos_hbm.at[pl.ds(base, RPW)], pos_v.at[pl.ds(0, RPW)])
    pltpu.sync_copy(
        pos_hbm.at[pl.ds(N + base, RPW)], pos_v.at[pl.ds(RPW, RPW)]
    )

    def _compute(v):
        xs = pos_v[pl.ds(v * 16, 16)]
        ys = pos_v[pl.ds(RPW + v * 16, 16)]
        fx = (((xs + 1.0) / _DENOM) * np.float32(WIDTH)).astype(jnp.int32)
        fy = (((ys + 1.0) / _DENOM) * np.float32(WIDTH)).astype(jnp.int32)
        idx_v[pl.ds(v * 16, 16)] = fx * WIDTH + fy

    HEAD = 2 * (C // 16)
    pl.loop(0, HEAD)(_compute)

    plsc.subcore_barrier()

    def _fire_rows(b, ch):
        for s in range(C // 16):
            ivec = idx_v[pl.ds(ch * C + s * 16, 16)]
            for k in range(16):
                pltpu.async_copy(
                    _row(ivec[k]), rows[b].at[s * 16 + k], gsem[b]
                )

    def _drain_rows(b):
        pltpu.make_async_copy(
            out_hbm.at[pl.ds(base, C)], rows[b], gsem[b]
        ).wait()

    def _scatter(b, ch):
        pltpu.async_copy(
            rows[b], out_hbm.at[pl.ds(base + ch * C, C)], ssem[b]
        )

    def _wait_scatter(b, ch):
        pltpu.make_async_copy(
            rows[b], out_hbm.at[pl.ds(base + ch * C, C)], ssem[b]
        ).wait()

    _fire_rows(0, 0)

    pl.loop(HEAD, NVEC)(_compute)

    @pl.loop(0, NCH, step=2)
    def _move(ch0):
        for b in range(2):
            ch = ch0 + b
            b1 = 1 - b
            nxt = ch + 1

            @pl.when(nxt < NCH)
            def _prefetch():
                @pl.when(nxt >= 2)
                def _reclaim():
                    _wait_scatter(b1, nxt - 2)

                _fire_rows(b1, nxt)

            _drain_rows(b)
            _scatter(b, ch)

    _wait_scatter((NCH - 2) % 2, NCH - 2)
    _wait_scatter((NCH - 1) % 2, NCH - 1)


@jax.jit
def _sc_embed(pos_tflat, table_flat):
    mesh = plsc.VectorSubcoreMesh(
        core_axis_name="c", subcore_axis_name="s", num_cores=NC, num_subcores=NS
    )
    return pl.kernel(
        _sc_body,
        out_type=jax.ShapeDtypeStruct((N, HIDDEN), jnp.float32),
        mesh=mesh,
        scratch_types=[
            pltpu.VMEM_SHARED((NUM_POS * HIDDEN,), jnp.float32),
            pltpu.VMEM((2 * RPW,), jnp.float32),
            pltpu.VMEM((RPW,), jnp.int32),
            pltpu.VMEM((C, HIDDEN), jnp.float32),
            pltpu.VMEM((C, HIDDEN), jnp.float32),
            pltpu.SemaphoreType.DMA,
            pltpu.SemaphoreType.DMA,
            pltpu.SemaphoreType.DMA,
            pltpu.SemaphoreType.DMA,
        ],
        compiler_params=pltpu.CompilerParams(needs_layout_passes=False),
    )(pos_tflat, table_flat)


def kernel(pos, positional_embeddings):
    pos_tflat = pos.T.reshape(2 * N)
    table_flat = positional_embeddings.reshape(NUM_POS * HIDDEN)
    out = _sc_embed(pos_tflat, table_flat)
    return out.reshape(1, N, HIDDEN)


# speedup vs baseline: 1.4106x; 1.0109x over previous
import jax
import jax.numpy as jnp
import numpy as np
from jax import lax
from jax.experimental import pallas as pl
from jax.experimental.pallas import tpu as pltpu
from jax.experimental.pallas import tpu_sc as plsc

HIDDEN = 768
NUM_POS = 577
WIDTH = 24
N = 131072

NC = 2
NS = 16
NW = NC * NS
RPW = N // NW
C = 16
NCH = RPW // C
NVEC = RPW // 16

_DENOM = np.float32(2.0 + 1e-6)


def _sc_body(pos_hbm, table_hbm, out_hbm, table_sh, pos_v, idx_v, rows0,
             rows1, g0, g1, s0, s1):
    sid = lax.axis_index("s")
    wid = sid * NC + lax.axis_index("c")
    base = wid * RPW
    rows = (rows0, rows1)
    gsem = (g0, g1)
    ssem = (s0, s1)

    @pl.when(sid == 0)
    def _stage_table():
        pltpu.sync_copy(table_hbm, table_sh)

    def _row(i):
        return table_sh.at[pl.ds(i * HIDDEN, HIDDEN)]

    pltpu.sync_copy(pos_hbm.at[pl.ds(base, RPW)], pos_v.at[pl.ds(0, RPW)])
    pltpu.sync_copy(
        pos_hbm.at[pl.ds(N + base, RPW)], pos_v.at[pl.ds(RPW, RPW)]
    )

    def _compute(v):
        xs = pos_v[pl.ds(v * 16, 16)]
        ys = pos_v[pl.ds(RPW + v * 16, 16)]
        fx = (((xs + 1.0) / _DENOM) * np.float32(WIDTH)).astype(jnp.int32)
        fy = (((ys + 1.0) / _DENOM) * np.float32(WIDTH)).astype(jnp.int32)
        idx_v[pl.ds(v * 16, 16)] = fx * WIDTH + fy

    HEAD = 2 * (C // 16)
    pl.loop(0, HEAD)(_compute)

    plsc.subcore_barrier()

    def _fire_rows(b, ch):
        for s in range(C // 16):
            ivec = idx_v[pl.ds(ch * C + s * 16, 16)]
            for k in range(16):
                pltpu.async_copy(
                    _row(ivec[k]), rows[b].at[s * 16 + k], gsem[b]
                )

    def _drain_rows(b):
        pltpu.make_async_copy(
            out_hbm.at[pl.ds(base, C)], rows[b], gsem[b]
        ).wait()

    def _scatter(b, ch):
        pltpu.async_copy(
            rows[b], out_hbm.at[pl.ds(base + ch * C, C)], ssem[b]
        )

    def _wait_scatter(b, ch):
        pltpu.make_async_copy(
            rows[b], out_hbm.at[pl.ds(base + ch * C, C)], ssem[b]
        ).wait()

    _fire_rows(0, 0)

    pl.loop(HEAD, NVEC)(_compute)

    @pl.loop(0, NCH, step=2)
    def _move(ch0):
        for b in range(2):
            ch = ch0 + b
            b1 = 1 - b
            nxt = ch + 1

            @pl.when(nxt < NCH)
            def _prefetch():
                @pl.when(nxt >= 2)
                def _reclaim():
                    _wait_scatter(b1, nxt - 2)

                _fire_rows(b1, nxt)

            _drain_rows(b)
            _scatter(b, ch)

    _wait_scatter((NCH - 2) % 2, NCH - 2)
    _wait_scatter((NCH - 1) % 2, NCH - 1)


@jax.jit
def _sc_embed(pos_tflat, table_flat):
    mesh = plsc.VectorSubcoreMesh(
        core_axis_name="c", subcore_axis_name="s", num_cores=NC, num_subcores=NS
    )
    return pl.kernel(
        _sc_body,
        out_type=jax.ShapeDtypeStruct((N, HIDDEN), jnp.float32),
        mesh=mesh,
        scratch_types=[
            pltpu.VMEM_SHARED((NUM_POS * HIDDEN,), jnp.float32),
            pltpu.VMEM((2 * RPW,), jnp.float32),
            pltpu.VMEM((RPW,), jnp.int32),
            pltpu.VMEM((C, HIDDEN), jnp.float32),
            pltpu.VMEM((C, HIDDEN), jnp.float32),
            pltpu.SemaphoreType.DMA,
            pltpu.SemaphoreType.DMA,
            pltpu.SemaphoreType.DMA,
            pltpu.SemaphoreType.DMA,
        ],
        compiler_params=pltpu.CompilerParams(needs_layout_passes=False),
    )(pos_tflat, table_flat)


def kernel(pos, positional_embeddings):
    pos_tflat = pos.T.reshape(2 * N)
    table_flat = positional_embeddings.reshape(NUM_POS * HIDDEN)
    out = _sc_embed(pos_tflat, table_flat)
    return out.reshape(1, N, HIDDEN)
